# baseline (speedup 1.0000x reference)
import functools

import jax
import jax.numpy as jnp
from jax import lax
from jax.experimental import pallas as pl
from jax.experimental.pallas import tpu as pltpu
from jax.experimental.pallas import tpu_sc as plsc

NB_SC = 64
NB_TC = 8


def _tc_body(dp_ref, x_ref, out_ref):
    blk = dp_ref[...]
    nb, k3, a, d = blk.shape
    z = blk.reshape(nb, k3 * a, d) * x_ref[:, :, :]
    y = jnp.sum(z, axis=2)
    out_ref[...] = jnp.sum(y * y, axis=1).reshape(nb, 1, 1)


def _tc_call(dp, x3, off_b, n_b):
    B, K, A, D = dp.shape
    noff = off_b // NB_TC
    out = pl.pallas_call(
        _tc_body,
        grid=(n_b // NB_TC,),
        in_specs=[
            pl.BlockSpec((NB_TC, K, A, D), lambda b: (b + noff, 0, 0, 0)),
            pl.BlockSpec((NB_TC, 1, D), lambda b: (b + noff, 0, 0)),
        ],
        out_specs=pl.BlockSpec((NB_TC, 1, 1), lambda b: (b, 0, 0)),
        out_shape=jax.ShapeDtypeStruct((n_b, 1, 1), jnp.float32),
        compiler_params=pltpu.CompilerParams(
            dimension_semantics=("arbitrary",),
        ),
    )(dp, x3)
    return out.reshape(n_b)


def _sc_call(dp6, x4, nb_sc):
    ncp = nb_sc * 24
    nw = 32
    cpw = ncp // nw
    mesh = plsc.VectorSubcoreMesh(core_axis_name="c", subcore_axis_name="s")

    def body(dp6_hbm, x4_hbm, out_hbm, buf, xbuf, totbuf):
        c = lax.axis_index("c")
        s = lax.axis_index("s")
        w = s * 2 + c
        lanes = lax.iota(jnp.int32, 16)
        i0 = lanes // 8
        i2 = lanes % 8

        def cp_body(j, carry):
            cp = w * cpw + j
            b = cp // 24
            r = cp - b * 24
            kk = r // 8
            rp = r - kk * 8
            bt = b // 8
            bs = b - bt * 8
            pltpu.sync_copy(dp6_hbm.at[b, kk, pl.ds(2 * rp, 2)], buf)
            pltpu.sync_copy(x4_hbm.at[bt, :, bs, :], xbuf)
            acc = jnp.zeros((16,), jnp.float32)
            for ct in range(4):
                cti = jnp.full((16,), ct, jnp.int32)

                def d_body(dd, a, cti=cti):
                    ddi = jnp.full((16,), dd, jnp.int32)
                    g = plsc.load_gather(buf, [i0, cti, i2, ddi])
                    xg = plsc.load_gather(xbuf, [cti, ddi])
                    return a + g * xg

                acc = lax.fori_loop(0, 128, d_body, acc, unroll=8)
            totbuf[...] = acc * acc
            pltpu.sync_copy(totbuf, out_hbm.at[pl.ds(cp * 16, 16)])
            return carry

        lax.fori_loop(0, cpw, cp_body, 0)

    fn = pl.kernel(
        body,
        out_type=jax.ShapeDtypeStruct((ncp * 16,), jnp.float32),
        mesh=mesh,
        scratch_types=[
            pltpu.VMEM((2, 4, 8, 128), jnp.float32),
            pltpu.VMEM((4, 128), jnp.float32),
            pltpu.VMEM((16,), jnp.float32),
        ],
        compiler_params=pltpu.CompilerParams(needs_layout_passes=False),
    )
    out = fn(dp6, x4)
    return out.reshape(nb_sc, 24 * 16).sum(axis=1)


def kernel(x, der_desc_wrt_coord):
    B, A, D, K = der_desc_wrt_coord.shape
    dp = jnp.transpose(der_desc_wrt_coord, (0, 3, 1, 2))
    parts = []
    if NB_SC > 0:
        dp6 = dp.reshape(B, K, A // 8, 8, D // 128, 128).transpose(0, 1, 2, 4, 3, 5)
        x4 = x.reshape(B // 8, 8, D // 128, 128).transpose(0, 2, 1, 3)
        parts.append(_sc_call(dp6, x4, NB_SC))
    if NB_SC < B:
        x3 = x.reshape(B, 1, D)
        parts.append(_tc_call(dp, x3, NB_SC, B - NB_SC))
    return parts[0] if len(parts) == 1 else jnp.concatenate(parts)


# speedup vs baseline: 1.2193x; 1.2193x over previous
import functools

import jax
import jax.numpy as jnp
from jax import lax
from jax.experimental import pallas as pl
from jax.experimental.pallas import tpu as pltpu
from jax.experimental.pallas import tpu_sc as plsc

NB_SC = 64
NB_TC = 8


def _tc_body(dp_ref, x_ref, out_ref):
    blk = dp_ref[...]
    nb, k3, a, d = blk.shape
    z = blk.reshape(nb, k3 * a, d) * x_ref[:, :, :]
    y = jnp.sum(z, axis=2)
    out_ref[...] = jnp.sum(y * y, axis=1).reshape(nb, 1, 1)


def _tc_call(dp, x3, off_b, n_b):
    B, K, A, D = dp.shape
    noff = off_b // NB_TC
    out = pl.pallas_call(
        _tc_body,
        grid=(n_b // NB_TC,),
        in_specs=[
            pl.BlockSpec((NB_TC, K, A, D), lambda b: (b + noff, 0, 0, 0)),
            pl.BlockSpec((NB_TC, 1, D), lambda b: (b + noff, 0, 0)),
        ],
        out_specs=pl.BlockSpec((NB_TC, 1, 1), lambda b: (b, 0, 0)),
        out_shape=jax.ShapeDtypeStruct((n_b, 1, 1), jnp.float32),
        compiler_params=pltpu.CompilerParams(
            dimension_semantics=("arbitrary",),
        ),
    )(dp, x3)
    return out.reshape(n_b)


def _sc_call(dp6, x4, nb_sc):
    ncp = nb_sc * 24
    nw = 32
    cpw = ncp // nw
    mesh = plsc.VectorSubcoreMesh(core_axis_name="c", subcore_axis_name="s")

    def body(dp6_hbm, x4_hbm, out_hbm, buf0, buf1, xb0, xb1, totbuf, sem0, sem1):
        c = lax.axis_index("c")
        s = lax.axis_index("s")
        w = s * 2 + c
        lanes = lax.iota(jnp.int32, 16)
        i0 = lanes // 8
        i2 = lanes % 8

        def fire(cp, buf, xb, sem):
            cpc = jnp.minimum(cp, ncp - 1)
            b = cpc // 24
            r = cpc - b * 24
            kk = r // 8
            rp = r - kk * 8
            h1 = pltpu.async_copy(dp6_hbm.at[b, kk, pl.ds(2 * rp, 2)], buf, sem)
            h2 = pltpu.async_copy(x4_hbm.at[b // 8, :, b - (b // 8) * 8, :], xb, sem)
            return h1, h2

        def wait(buf, xb, sem):
            pltpu.make_async_copy(dp6_hbm.at[0, 0, pl.ds(0, 2)], buf, sem).wait()
            pltpu.make_async_copy(x4_hbm.at[0, :, 0, :], xb, sem).wait()

        def compute(buf, xb, cp):
            z = jnp.zeros((16,), jnp.float32)
            acc4 = (z, z, z, z)
            for ct in range(4):
                cti = jnp.full((16,), ct, jnp.int32)

                def d_body(t, a4, cti=cti):
                    dd = t * 4
                    outs = []
                    for i in range(4):
                        ddi = jnp.full((16,), dd + i, jnp.int32)
                        g = plsc.load_gather(buf, [i0, cti, i2, ddi])
                        xg = plsc.load_gather(xb, [cti, ddi])
                        outs.append(a4[i] + g * xg)
                    return tuple(outs)

                acc4 = lax.fori_loop(0, 32, d_body, acc4, unroll=4)
            acc = (acc4[0] + acc4[1]) + (acc4[2] + acc4[3])
            totbuf[...] = acc * acc
            pltpu.sync_copy(totbuf, out_hbm.at[pl.ds(cp * 16, 16)])

        fire(w * cpw, buf0, xb0, sem0)

        def pair_body(t, carry):
            cp0 = w * cpw + t * 2
            h1, h2 = fire(cp0 + 1, buf1, xb1, sem1)
            wait(buf0, xb0, sem0)
            compute(buf0, xb0, cp0)
            fire(cp0 + 2, buf0, xb0, sem0)
            h1.wait()
            h2.wait()
            compute(buf1, xb1, cp0 + 1)
            return carry

        lax.fori_loop(0, cpw // 2, pair_body, 0)
        wait(buf0, xb0, sem0)

    fn = pl.kernel(
        body,
        out_type=jax.ShapeDtypeStruct((ncp * 16,), jnp.float32),
        mesh=mesh,
        scratch_types=[
            pltpu.VMEM((2, 4, 8, 128), jnp.float32),
            pltpu.VMEM((2, 4, 8, 128), jnp.float32),
            pltpu.VMEM((4, 128), jnp.float32),
            pltpu.VMEM((4, 128), jnp.float32),
            pltpu.VMEM((16,), jnp.float32),
            pltpu.SemaphoreType.DMA,
            pltpu.SemaphoreType.DMA,
        ],
        compiler_params=pltpu.CompilerParams(needs_layout_passes=False),
    )
    out = fn(dp6, x4)
    return out.reshape(nb_sc, 24 * 16).sum(axis=1)


def kernel(x, der_desc_wrt_coord):
    B, A, D, K = der_desc_wrt_coord.shape
    dp = jnp.transpose(der_desc_wrt_coord, (0, 3, 1, 2))
    parts = []
    if NB_SC > 0:
        dp6 = dp.reshape(B, K, A // 8, 8, D // 128, 128).transpose(0, 1, 2, 4, 3, 5)
        x4 = x.reshape(B // 8, 8, D // 128, 128).transpose(0, 2, 1, 3)
        parts.append(_sc_call(dp6, x4, NB_SC))
    if NB_SC < B:
        x3 = x.reshape(B, 1, D)
        parts.append(_tc_call(dp, x3, NB_SC, B - NB_SC))
    return parts[0] if len(parts) == 1 else jnp.concatenate(parts)


# speedup vs baseline: 3.0524x; 2.5035x over previous
import functools

import jax
import jax.numpy as jnp
from jax import lax
from jax.experimental import pallas as pl
from jax.experimental.pallas import tpu as pltpu
from jax.experimental.pallas import tpu_sc as plsc

NB_SC = 64
NB_TC = 8


def _tc_body(dp_ref, x_ref, out_ref):
    blk = dp_ref[...]
    nb, k3, a, d = blk.shape
    z = blk.reshape(nb, k3 * a, d) * x_ref[:, :, :]
    y = jnp.sum(z, axis=2)
    out_ref[...] = jnp.sum(y * y, axis=1).reshape(nb, 1, 1)


def _tc_call(dp, x3, off_b, n_b):
    B, K, A, D = dp.shape
    noff = off_b // NB_TC
    out = pl.pallas_call(
        _tc_body,
        grid=(n_b // NB_TC,),
        in_specs=[
            pl.BlockSpec((NB_TC, K, A, D), lambda b: (b + noff, 0, 0, 0)),
            pl.BlockSpec((NB_TC, 1, D), lambda b: (b + noff, 0, 0)),
        ],
        out_specs=pl.BlockSpec((NB_TC, 1, 1), lambda b: (b, 0, 0)),
        out_shape=jax.ShapeDtypeStruct((n_b, 1, 1), jnp.float32),
        compiler_params=pltpu.CompilerParams(
            dimension_semantics=("arbitrary",),
        ),
    )(dp, x3)
    return out.reshape(n_b)


def _sc_call(dpf, x4, nb_sc):
    ncp = nb_sc * 24
    nw = 32
    cpw = ncp // nw
    mesh = plsc.VectorSubcoreMesh(core_axis_name="c", subcore_axis_name="s")

    def body(dpf_hbm, x4_hbm, out_hbm, buf0, buf1, xb0, xb1, totbuf, sem0, sem1):
        c = lax.axis_index("c")
        s = lax.axis_index("s")
        w = s * 2 + c
        lanes = lax.iota(jnp.int32, 16)
        e15 = (lanes == 15).astype(jnp.float32)

        def fire(cp, buf, xb, sem):
            cpc = jnp.minimum(cp, ncp - 1)
            b = cpc // 24
            h1 = pltpu.async_copy(dpf_hbm.at[pl.ds(cpc * 8192, 8192)], buf, sem)
            h2 = pltpu.async_copy(x4_hbm.at[b // 8, :, b - (b // 8) * 8, :], xb, sem)
            return h1, h2

        def wait(buf, xb, sem):
            pltpu.make_async_copy(dpf_hbm.at[pl.ds(0, 8192)], buf, sem).wait()
            pltpu.make_async_copy(x4_hbm.at[0, :, 0, :], xb, sem).wait()

        def compute(buf, xb, cp):
            accs = [jnp.zeros((16,), jnp.float32)] * 16
            for ct in range(4):
                for v in range(8):
                    xv = xb[ct, pl.ds(v * 16, 16)]
                    for rt in range(2):
                        for sr in range(8):
                            a = rt * 8 + sr
                            off = rt * 4096 + ct * 1024 + sr * 128 + v * 16
                            accs[a] = accs[a] + buf[pl.ds(off, 16)] * xv
            tot = jnp.zeros((16,), jnp.float32)
            for a in range(16):
                cs = plsc.cumsum(accs[a])
                tot = tot + cs * cs * e15
            totbuf[...] = tot
            pltpu.sync_copy(totbuf, out_hbm.at[pl.ds(cp * 16, 16)])

        fire(w * cpw, buf0, xb0, sem0)

        def pair_body(t, carry):
            cp0 = w * cpw + t * 2
            h1, h2 = fire(cp0 + 1, buf1, xb1, sem1)
            wait(buf0, xb0, sem0)
            compute(buf0, xb0, cp0)
            fire(cp0 + 2, buf0, xb0, sem0)
            h1.wait()
            h2.wait()
            compute(buf1, xb1, cp0 + 1)
            return carry

        lax.fori_loop(0, cpw // 2, pair_body, 0)
        wait(buf0, xb0, sem0)

    fn = pl.kernel(
        body,
        out_type=jax.ShapeDtypeStruct((ncp * 16,), jnp.float32),
        mesh=mesh,
        scratch_types=[
            pltpu.VMEM((8192,), jnp.float32),
            pltpu.VMEM((8192,), jnp.float32),
            pltpu.VMEM((4, 128), jnp.float32),
            pltpu.VMEM((4, 128), jnp.float32),
            pltpu.VMEM((16,), jnp.float32),
            pltpu.SemaphoreType.DMA,
            pltpu.SemaphoreType.DMA,
        ],
        compiler_params=pltpu.CompilerParams(needs_layout_passes=False),
    )
    out = fn(dpf, x4)
    return out.reshape(nb_sc, 24 * 16).sum(axis=1)


def kernel(x, der_desc_wrt_coord):
    B, A, D, K = der_desc_wrt_coord.shape
    dp = jnp.transpose(der_desc_wrt_coord, (0, 3, 1, 2))
    parts = []
    if NB_SC > 0:
        dp6 = dp.reshape(B, K, A // 8, 8, D // 128, 128).transpose(0, 1, 2, 4, 3, 5)
        dpf = dp6.reshape(-1)
        x4 = x.reshape(B // 8, 8, D // 128, 128).transpose(0, 2, 1, 3)
        parts.append(_sc_call(dpf, x4, NB_SC))
    if NB_SC < B:
        x3 = x.reshape(B, 1, D)
        parts.append(_tc_call(dp, x3, NB_SC, B - NB_SC))
    return parts[0] if len(parts) == 1 else jnp.concatenate(parts)


# speedup vs baseline: 7.1512x; 2.3428x over previous
import functools

import jax
import jax.numpy as jnp
from jax import lax
from jax.experimental import pallas as pl
from jax.experimental.pallas import tpu as pltpu
from jax.experimental.pallas import tpu_sc as plsc

NB_SC = 8
NB_TC = 8


def _tc_body(dp_ref, x_ref, out_ref):
    blk = dp_ref[...]
    nb, k3, a, d = blk.shape
    z = blk.reshape(nb, k3 * a, d) * x_ref[:, :, :]
    y = jnp.sum(z, axis=2)
    out_ref[...] = jnp.sum(y * y, axis=1).reshape(nb, 1, 1)


def _tc_call(dp, x3, off_b, n_b):
    B, K, A, D = dp.shape
    noff = off_b // NB_TC
    out = pl.pallas_call(
        _tc_body,
        grid=(n_b // NB_TC,),
        in_specs=[
            pl.BlockSpec((NB_TC, K, A, D), lambda b: (b + noff, 0, 0, 0)),
            pl.BlockSpec((NB_TC, 1, D), lambda b: (b + noff, 0, 0)),
        ],
        out_specs=pl.BlockSpec((NB_TC, 1, 1), lambda b: (b, 0, 0)),
        out_shape=jax.ShapeDtypeStruct((n_b, 1, 1), jnp.float32),
        compiler_params=pltpu.CompilerParams(
            dimension_semantics=("arbitrary",),
        ),
    )(dp, x3)
    return out.reshape(n_b)


def _sc_call(dpf, x4, nb_sc):
    ncp = nb_sc * 24
    nw = 32
    cpw = ncp // nw
    mesh = plsc.VectorSubcoreMesh(core_axis_name="c", subcore_axis_name="s")

    def body(dpf_hbm, x4_hbm, out_hbm, buf0, buf1, xb0, xb1, totbuf, sem0, sem1):
        c = lax.axis_index("c")
        s = lax.axis_index("s")
        w = s * 2 + c
        lanes = lax.iota(jnp.int32, 16)
        e15 = (lanes == 15).astype(jnp.float32)

        def fire(cp, buf, xb, sem):
            cpc = jnp.minimum(cp, ncp - 1)
            b = cpc // 24
            h1 = pltpu.async_copy(dpf_hbm.at[pl.ds(cpc * 8192, 8192)], buf, sem)
            h2 = pltpu.async_copy(x4_hbm.at[b // 8, :, b - (b // 8) * 8, :], xb, sem)
            return h1, h2

        def wait(buf, xb, sem):
            pltpu.make_async_copy(dpf_hbm.at[pl.ds(0, 8192)], buf, sem).wait()
            pltpu.make_async_copy(x4_hbm.at[0, :, 0, :], xb, sem).wait()

        def compute(buf, xb, cp):
            accs = [jnp.zeros((16,), jnp.float32)] * 16
            for ct in range(4):
                for v in range(8):
                    xv = xb[ct, pl.ds(v * 16, 16)]
                    for rt in range(2):
                        for sr in range(8):
                            a = rt * 8 + sr
                            off = rt * 4096 + ct * 1024 + sr * 128 + v * 16
                            accs[a] = accs[a] + buf[pl.ds(off, 16)] * xv
            tot = jnp.zeros((16,), jnp.float32)
            for a in range(16):
                cs = plsc.cumsum(accs[a])
                tot = tot + cs * cs * e15
            totbuf[...] = tot
            pltpu.sync_copy(totbuf, out_hbm.at[pl.ds(cp * 16, 16)])

        fire(w * cpw, buf0, xb0, sem0)

        def pair_body(t, carry):
            cp0 = w * cpw + t * 2
            h1, h2 = fire(cp0 + 1, buf1, xb1, sem1)
            wait(buf0, xb0, sem0)
            compute(buf0, xb0, cp0)
            fire(cp0 + 2, buf0, xb0, sem0)
            h1.wait()
            h2.wait()
            compute(buf1, xb1, cp0 + 1)
            return carry

        lax.fori_loop(0, cpw // 2, pair_body, 0)
        wait(buf0, xb0, sem0)

    fn = pl.kernel(
        body,
        out_type=jax.ShapeDtypeStruct((ncp * 16,), jnp.float32),
        mesh=mesh,
        scratch_types=[
            pltpu.VMEM((8192,), jnp.float32),
            pltpu.VMEM((8192,), jnp.float32),
            pltpu.VMEM((4, 128), jnp.float32),
            pltpu.VMEM((4, 128), jnp.float32),
            pltpu.VMEM((16,), jnp.float32),
            pltpu.SemaphoreType.DMA,
            pltpu.SemaphoreType.DMA,
        ],
        compiler_params=pltpu.CompilerParams(needs_layout_passes=False),
    )
    out = fn(dpf, x4)
    return out.reshape(nb_sc, 24 * 16).sum(axis=1)


def kernel(x, der_desc_wrt_coord):
    B, A, D, K = der_desc_wrt_coord.shape
    dp = jnp.transpose(der_desc_wrt_coord, (0, 3, 1, 2))
    parts = []
    if NB_SC > 0:
        dp6 = dp.reshape(B, K, A // 8, 8, D // 128, 128).transpose(0, 1, 2, 4, 3, 5)
        dpf = dp6.reshape(-1)
        x4 = x.reshape(B // 8, 8, D // 128, 128).transpose(0, 2, 1, 3)
        parts.append(_sc_call(dpf, x4, NB_SC))
    if NB_SC < B:
        x3 = x.reshape(B, 1, D)
        parts.append(_tc_call(dp, x3, NB_SC, B - NB_SC))
    return parts[0] if len(parts) == 1 else jnp.concatenate(parts)


# speedup vs baseline: 7.1527x; 1.0002x over previous
import functools

import jax
import jax.numpy as jnp
from jax import lax
from jax.experimental import pallas as pl
from jax.experimental.pallas import tpu as pltpu
from jax.experimental.pallas import tpu_sc as plsc

NB_SC = 8
NB_TC = 8


def _tc_body(dp_ref, x_ref, out_ref):
    blk = dp_ref[...]
    nb, k3, a, d = blk.shape
    z = blk.reshape(nb, k3 * a, d) * x_ref[:, :, :]
    y = jnp.sum(z, axis=2)
    out_ref[...] = jnp.sum(y * y, axis=1).reshape(nb, 1, 1)


def _tc_call(dp, x3, off_b, n_b):
    B, K, A, D = dp.shape
    noff = off_b // NB_TC
    out = pl.pallas_call(
        _tc_body,
        grid=(n_b // NB_TC,),
        in_specs=[
            pl.BlockSpec((NB_TC, K, A, D), lambda b: (b + noff, 0, 0, 0)),
            pl.BlockSpec((NB_TC, 1, D), lambda b: (b + noff, 0, 0)),
        ],
        out_specs=pl.BlockSpec((NB_TC, 1, 1), lambda b: (b, 0, 0)),
        out_shape=jax.ShapeDtypeStruct((n_b, 1, 1), jnp.float32),
        compiler_params=pltpu.CompilerParams(
            dimension_semantics=("arbitrary",),
            skip_device_barrier=True,
        ),
    )(dp, x3)
    return out.reshape(n_b)


def _sc_call(dpf, x4, nb_sc):
    ncp = nb_sc * 24
    nw = 32
    cpw = ncp // nw
    mesh = plsc.VectorSubcoreMesh(core_axis_name="c", subcore_axis_name="s")

    def body(dpf_hbm, x4_hbm, out_hbm, buf0, buf1, xb0, xb1, totbuf, sem0, sem1):
        c = lax.axis_index("c")
        s = lax.axis_index("s")
        w = s * 2 + c
        lanes = lax.iota(jnp.int32, 16)
        e15 = (lanes == 15).astype(jnp.float32)

        def fire(cp, buf, xb, sem):
            cpc = jnp.minimum(cp, ncp - 1)
            b = cpc // 24
            h1 = pltpu.async_copy(dpf_hbm.at[pl.ds(cpc * 8192, 8192)], buf, sem)
            h2 = pltpu.async_copy(x4_hbm.at[b // 8, :, b - (b // 8) * 8, :], xb, sem)
            return h1, h2

        def wait(buf, xb, sem):
            pltpu.make_async_copy(dpf_hbm.at[pl.ds(0, 8192)], buf, sem).wait()
            pltpu.make_async_copy(x4_hbm.at[0, :, 0, :], xb, sem).wait()

        def compute(buf, xb, cp):
            accs = [jnp.zeros((16,), jnp.float32)] * 16
            for ct in range(4):
                for v in range(8):
                    xv = xb[ct, pl.ds(v * 16, 16)]
                    for rt in range(2):
                        for sr in range(8):
                            a = rt * 8 + sr
                            off = rt * 4096 + ct * 1024 + sr * 128 + v * 16
                            accs[a] = accs[a] + buf[pl.ds(off, 16)] * xv
            tot = jnp.zeros((16,), jnp.float32)
            for a in range(16):
                cs = plsc.cumsum(accs[a])
                tot = tot + cs * cs * e15
            totbuf[...] = tot
            pltpu.sync_copy(totbuf, out_hbm.at[pl.ds(cp * 16, 16)])

        fire(w * cpw, buf0, xb0, sem0)

        def pair_body(t, carry):
            cp0 = w * cpw + t * 2
            h1, h2 = fire(cp0 + 1, buf1, xb1, sem1)
            wait(buf0, xb0, sem0)
            compute(buf0, xb0, cp0)
            fire(cp0 + 2, buf0, xb0, sem0)
            h1.wait()
            h2.wait()
            compute(buf1, xb1, cp0 + 1)
            return carry

        lax.fori_loop(0, cpw // 2, pair_body, 0)
        wait(buf0, xb0, sem0)

    fn = pl.kernel(
        body,
        out_type=jax.ShapeDtypeStruct((ncp * 16,), jnp.float32),
        mesh=mesh,
        scratch_types=[
            pltpu.VMEM((8192,), jnp.float32),
            pltpu.VMEM((8192,), jnp.float32),
            pltpu.VMEM((4, 128), jnp.float32),
            pltpu.VMEM((4, 128), jnp.float32),
            pltpu.VMEM((16,), jnp.float32),
            pltpu.SemaphoreType.DMA,
            pltpu.SemaphoreType.DMA,
        ],
        compiler_params=pltpu.CompilerParams(
            needs_layout_passes=False,
            skip_device_barrier=True,
        ),
    )
    out = fn(dpf, x4)
    return out.reshape(nb_sc, 24 * 16).sum(axis=1)


def kernel(x, der_desc_wrt_coord):
    B, A, D, K = der_desc_wrt_coord.shape
    dp = jnp.transpose(der_desc_wrt_coord, (0, 3, 1, 2))
    parts = []
    if NB_SC > 0:
        dp6 = dp.reshape(B, K, A // 8, 8, D // 128, 128).transpose(0, 1, 2, 4, 3, 5)
        dpf = dp6.reshape(-1)
        x4 = x.reshape(B // 8, 8, D // 128, 128).transpose(0, 2, 1, 3)
        parts.append(_sc_call(dpf, x4, NB_SC))
    if NB_SC < B:
        x3 = x.reshape(B, 1, D)
        parts.append(_tc_call(dp, x3, NB_SC, B - NB_SC))
    return parts[0] if len(parts) == 1 else jnp.concatenate(parts)
